# Initial kernel scaffold; baseline (speedup 1.0000x reference)
import jax
import jax.numpy as jnp
from jax.experimental import pallas as pl


def kernel(x, edge_index, batch, W1, b1, W2, b2, Wl, bl, Wo, bo):
    raise NotImplementedError("write your pallas kernel here")


# speedup vs baseline: 6.9309x; 6.9309x over previous
import functools

import jax
import jax.numpy as jnp
from jax import lax
from jax.experimental import pallas as pl
from jax.experimental.pallas import tpu as pltpu
from jax.experimental.pallas import tpu_sc as plsc

N = 10000
NPAD = 10240
D = 128
E = 320000
EPAD = 327680
NW = 32
EPT = EPAD // NW
CH = 128
NCH = EPT // CH
RPT = NPAD // 16
NZC = RPT // CH
DEGW = 128
NG = 64
BLK = 512
NB = NPAD // BLK
NC = 16
NEG = 0.01


@functools.cache
def _build_deg_kernel():
    mesh = plsc.VectorSubcoreMesh(core_axis_name="c", subcore_axis_name="s")
    return pl.kernel(
        _deg_body,
        mesh=mesh,
        out_type=jax.ShapeDtypeStruct((2 * NPAD, DEGW), jnp.float32),
        scratch_types=[
            pltpu.VMEM((CH,), jnp.int32),
            pltpu.VMEM((CH, DEGW), jnp.float32),
            pltpu.VMEM_SHARED((NPAD, DEGW), jnp.float32),
        ],
    )


def _deg_body(dst_hbm, ones_hbm, zrow_hbm, out_hbm, didx, ones_v, acc_sh):
    c = lax.axis_index("c")
    s = lax.axis_index("s")
    wid = s * 2 + c

    pltpu.sync_copy(ones_hbm, ones_v)

    base_r = s * RPT

    def zinit(k, _):
        pltpu.sync_copy(zrow_hbm, acc_sh.at[pl.ds(base_r + k * CH, CH)])
        return 0

    lax.fori_loop(0, NZC, zinit, 0)
    plsc.subcore_barrier()

    ebase = wid * EPT

    def body(ch, _):
        pltpu.sync_copy(dst_hbm.at[pl.ds(ebase + ch * CH, CH)], didx)
        pltpu.sync_copy(ones_v, acc_sh.at[didx], add=True)
        return 0

    lax.fori_loop(0, NCH, body, 0)
    plsc.subcore_barrier()

    def wb(k, _):
        r = base_r + k * CH
        pltpu.sync_copy(acc_sh.at[pl.ds(r, CH)],
                        out_hbm.at[pl.ds(c * NPAD + r, CH)])
        return 0

    lax.fori_loop(0, NZC, wb, 0)


@functools.cache
def _build_agg_kernel():
    mesh = plsc.VectorSubcoreMesh(core_axis_name="c", subcore_axis_name="s")
    return pl.kernel(
        _agg_body,
        mesh=mesh,
        out_type=jax.ShapeDtypeStruct((2 * NPAD, D), jnp.float32),
        scratch_types=[
            pltpu.VMEM((CH,), jnp.int32),
            pltpu.VMEM((CH,), jnp.int32),
            pltpu.VMEM((CH, D), jnp.float32),
            pltpu.VMEM_SHARED((NPAD, D), jnp.float32),
            pltpu.SemaphoreType.DMA,
        ],
    )


def _agg_body(y_hbm, src_hbm, dst_hbm, zrows_hbm, out_hbm, sidx, didx, rows,
              acc_sh, sem):
    c = lax.axis_index("c")
    s = lax.axis_index("s")
    wid = s * 2 + c

    base_r = s * RPT

    def zinit(k, _):
        pltpu.sync_copy(zrows_hbm, acc_sh.at[pl.ds(base_r + k * CH, CH)])
        return 0

    lax.fori_loop(0, NZC, zinit, 0)
    plsc.subcore_barrier()

    ebase = wid * EPT

    def body(ch, _):
        eb = ebase + ch * CH
        pltpu.sync_copy(src_hbm.at[pl.ds(eb, CH)], sidx)
        pltpu.sync_copy(dst_hbm.at[pl.ds(eb, CH)], didx)
        pltpu.async_copy(y_hbm.at[sidx], rows, sem).wait()
        pltpu.sync_copy(rows, acc_sh.at[didx], add=True)
        return 0

    lax.fori_loop(0, NCH, body, 0)
    plsc.subcore_barrier()

    def wb(k, _):
        r = base_r + k * CH
        pltpu.sync_copy(acc_sh.at[pl.ds(r, CH)],
                        out_hbm.at[pl.ds(c * NPAD + r, CH)])
        return 0

    lax.fori_loop(0, NZC, wb, 0)


def _dinv_block(deg_ref, i):
    deg = deg_ref[0, :, 0:1] + deg_ref[1, :, 0:1] + 1.0
    rows = i * BLK + lax.broadcasted_iota(jnp.int32, (BLK, 1), 0)
    return jnp.where(rows < N, lax.rsqrt(deg), 0.0)


def _mm_scale_body(x_ref, w_ref, deg_ref, y_ref):
    i = pl.program_id(0)
    dinv = _dinv_block(deg_ref, i)
    y_ref[...] = dinv * jnp.dot(x_ref[...], w_ref[...],
                                preferred_element_type=jnp.float32)


def _layer2_body(agg_ref, y_ref, deg_ref, b1_ref, w2_ref, out_ref):
    i = pl.program_id(0)
    dinv = _dinv_block(deg_ref, i)
    h = dinv * (agg_ref[0] + agg_ref[1] + y_ref[...]) + b1_ref[...]
    h = jnp.where(h > 0, h, NEG * h)
    out_ref[...] = dinv * jnp.dot(h, w2_ref[...],
                                  preferred_element_type=jnp.float32)


def _pool_head_body(agg_ref, y_ref, deg_ref, b2_ref, batch_ref,
                    wl_ref, bl_ref, wo_ref, bo_ref,
                    logits_ref, probs_ref, embeds_ref, acc_ref):
    i = pl.program_id(0)

    @pl.when(i == 0)
    def _():
        acc_ref[...] = jnp.full((NG, D), -jnp.inf, jnp.float32)

    dinv = _dinv_block(deg_ref, i)
    h = dinv * (agg_ref[0] + agg_ref[1] + y_ref[...]) + b2_ref[...]

    b = batch_ref[...]
    g_lo = jnp.min(b)
    g_hi = jnp.minimum(jnp.max(b), NG - 1)

    def seg(g, _):
        vals = jnp.where(b == g, h, -jnp.inf)
        m = jnp.max(vals, axis=0, keepdims=True)
        acc_ref[pl.ds(g, 1), :] = jnp.maximum(acc_ref[pl.ds(g, 1), :], m)
        return 0

    lax.fori_loop(g_lo, g_hi + 1, seg, 0)

    @pl.when(i == NB - 1)
    def _():
        pooled = acc_ref[...]
        embeds = jnp.where(pooled == -jnp.inf, 0.0, pooled)
        g1 = jnp.dot(embeds, wl_ref[...],
                     preferred_element_type=jnp.float32) + bl_ref[...]
        g1 = jnp.where(g1 > 0, g1, NEG * g1)
        logits = jnp.dot(g1, wo_ref[...],
                         preferred_element_type=jnp.float32) + bo_ref[...]
        m = jnp.max(logits, axis=-1, keepdims=True)
        ex = jnp.exp(logits - m)
        probs = ex / jnp.sum(ex, axis=-1, keepdims=True)
        logits_ref[...] = logits
        probs_ref[...] = probs
        embeds_ref[...] = embeds


def _mm_scale(x_pad, W1, deg2):
    return pl.pallas_call(
        _mm_scale_body,
        grid=(NB,),
        in_specs=[
            pl.BlockSpec((BLK, D), lambda i: (i, 0)),
            pl.BlockSpec((D, D), lambda i: (0, 0)),
            pl.BlockSpec((2, BLK, DEGW), lambda i: (0, i, 0)),
        ],
        out_specs=pl.BlockSpec((BLK, D), lambda i: (i, 0)),
        out_shape=jax.ShapeDtypeStruct((NPAD, D), jnp.float32),
    )(x_pad, W1, deg2)


def _layer2(agg1, y1, deg2, b1r, W2):
    return pl.pallas_call(
        _layer2_body,
        grid=(NB,),
        in_specs=[
            pl.BlockSpec((2, BLK, D), lambda i: (0, i, 0)),
            pl.BlockSpec((BLK, D), lambda i: (i, 0)),
            pl.BlockSpec((2, BLK, DEGW), lambda i: (0, i, 0)),
            pl.BlockSpec((1, D), lambda i: (0, 0)),
            pl.BlockSpec((D, D), lambda i: (0, 0)),
        ],
        out_specs=pl.BlockSpec((BLK, D), lambda i: (i, 0)),
        out_shape=jax.ShapeDtypeStruct((NPAD, D), jnp.float32),
    )(agg1, y1, deg2, b1r, W2)


def _pool_head(agg2, y2, deg2, b2r, batch_bc, Wl, blr, Wo, bor):
    return pl.pallas_call(
        _pool_head_body,
        grid=(NB,),
        in_specs=[
            pl.BlockSpec((2, BLK, D), lambda i: (0, i, 0)),
            pl.BlockSpec((BLK, D), lambda i: (i, 0)),
            pl.BlockSpec((2, BLK, DEGW), lambda i: (0, i, 0)),
            pl.BlockSpec((1, D), lambda i: (0, 0)),
            pl.BlockSpec((BLK, D), lambda i: (i, 0)),
            pl.BlockSpec((D, D), lambda i: (0, 0)),
            pl.BlockSpec((1, D), lambda i: (0, 0)),
            pl.BlockSpec((D, NC), lambda i: (0, 0)),
            pl.BlockSpec((1, NC), lambda i: (0, 0)),
        ],
        out_specs=[
            pl.BlockSpec((NG, NC), lambda i: (0, 0)),
            pl.BlockSpec((NG, NC), lambda i: (0, 0)),
            pl.BlockSpec((NG, D), lambda i: (0, 0)),
        ],
        out_shape=[
            jax.ShapeDtypeStruct((NG, NC), jnp.float32),
            jax.ShapeDtypeStruct((NG, NC), jnp.float32),
            jax.ShapeDtypeStruct((NG, D), jnp.float32),
        ],
        scratch_shapes=[pltpu.VMEM((NG, D), jnp.float32)],
    )(agg2, y2, deg2, b2r, batch_bc, Wl, blr, Wo, bor)


def kernel(x, edge_index, batch, W1, b1, W2, b2, Wl, bl, Wo, bo):
    src = edge_index[0].astype(jnp.int32)
    dst = edge_index[1].astype(jnp.int32)
    pad_idx = jnp.full((EPAD - E,), N, jnp.int32)
    srcp = jnp.concatenate([src, pad_idx])
    dstp = jnp.concatenate([dst, pad_idx])
    x_pad = jnp.pad(x, ((0, NPAD - N), (0, 0)))
    batch_pad = jnp.pad(batch.astype(jnp.int32), (0, NPAD - N),
                        constant_values=127)
    batch_bc = jnp.broadcast_to(batch_pad[:, None], (NPAD, D))
    b1r = b1.reshape(1, D)
    b2r = b2.reshape(1, D)
    blr = bl.reshape(1, D)
    bor = bo.reshape(1, NC)

    ones_row = jnp.ones((CH, DEGW), jnp.float32)
    zrow = jnp.zeros((CH, DEGW), jnp.float32)
    zrows = jnp.zeros((CH, D), jnp.float32)

    deg_fn = _build_deg_kernel()
    agg_fn = _build_agg_kernel()
    deg2 = deg_fn(dstp, ones_row, zrow).reshape(2, NPAD, DEGW)
    y1 = _mm_scale(x_pad, W1, deg2)
    agg1 = agg_fn(y1, srcp, dstp, zrows).reshape(2, NPAD, D)
    y2 = _layer2(agg1, y1, deg2, b1r, W2)
    agg2 = agg_fn(y2, srcp, dstp, zrows).reshape(2, NPAD, D)
    logits, probs, embeds = _pool_head(agg2, y2, deg2, b2r, batch_bc,
                                       Wl, blr, Wo, bor)
    return (logits, probs, embeds)


# speedup vs baseline: 8.4847x; 1.2242x over previous
import functools

import jax
import jax.numpy as jnp
from jax import lax
from jax.experimental import pallas as pl
from jax.experimental.pallas import tpu as pltpu
from jax.experimental.pallas import tpu_sc as plsc

N = 10000
NPAD = 10240
D = 128
E = 320000
EPAD = 327680
NW = 32
EPT = EPAD // NW
CH = 128
NCH = EPT // CH
NCH0 = 80
NCH1 = 80
RPT = NPAD // 16
NZC = RPT // CH
DEGW = 128
NG = 64
BLK = 512
NB = NPAD // BLK
NC = 16
NEG = 0.01


@functools.cache
def _build_deg_kernel():
    mesh = plsc.VectorSubcoreMesh(core_axis_name="c", subcore_axis_name="s")
    return pl.kernel(
        _deg_body,
        mesh=mesh,
        out_type=jax.ShapeDtypeStruct((2 * NPAD, DEGW), jnp.float32),
        scratch_types=[
            pltpu.VMEM((CH,), jnp.int32),
            pltpu.VMEM((CH, DEGW), jnp.float32),
            pltpu.VMEM_SHARED((NPAD, DEGW), jnp.float32),
        ],
    )


def _deg_body(dst_hbm, ones_hbm, zrow_hbm, out_hbm, didx, ones_v, acc_sh):
    c = lax.axis_index("c")
    s = lax.axis_index("s")
    wid = s * 2 + c

    pltpu.sync_copy(ones_hbm, ones_v)

    base_r = s * RPT

    def zinit(k, _):
        pltpu.sync_copy(zrow_hbm, acc_sh.at[pl.ds(base_r + k * CH, CH)])
        return 0

    lax.fori_loop(0, NZC, zinit, 0)
    plsc.subcore_barrier()

    ebase = wid * EPT

    def body(ch, _):
        pltpu.sync_copy(dst_hbm.at[pl.ds(ebase + ch * CH, CH)], didx)
        pltpu.sync_copy(ones_v, acc_sh.at[didx], add=True)
        return 0

    lax.fori_loop(0, NCH, body, 0)
    plsc.subcore_barrier()

    def wb(k, _):
        r = base_r + k * CH
        pltpu.sync_copy(acc_sh.at[pl.ds(r, CH)],
                        out_hbm.at[pl.ds(c * NPAD + r, CH)])
        return 0

    lax.fori_loop(0, NZC, wb, 0)


@functools.cache
def _build_agg_kernel():
    mesh = plsc.VectorSubcoreMesh(core_axis_name="c", subcore_axis_name="s")
    return pl.kernel(
        _agg_body,
        mesh=mesh,
        out_type=jax.ShapeDtypeStruct((2 * NPAD, D), jnp.float32),
        scratch_types=[
            pltpu.VMEM((CH,), jnp.int32),
            pltpu.VMEM((CH,), jnp.int32),
            pltpu.VMEM((CH,), jnp.int32),
            pltpu.VMEM((CH,), jnp.int32),
            pltpu.VMEM((CH, D), jnp.float32),
            pltpu.VMEM((CH, D), jnp.float32),
            pltpu.VMEM_SHARED((NPAD, D), jnp.float32),
            pltpu.SemaphoreType.DMA,
            pltpu.SemaphoreType.DMA,
        ],
    )


def _agg_body(y_hbm, src_hbm, dst_hbm, zrows_hbm, out_hbm,
              sidx0, didx0, sidx1, didx1, rows0, rows1, acc_sh, sem0, sem1):
    c = lax.axis_index("c")
    s = lax.axis_index("s")

    base_r = s * RPT

    def zinit(k, _):
        pltpu.sync_copy(zrows_hbm, acc_sh.at[pl.ds(base_r + k * CH, CH)])
        return 0

    lax.fori_loop(0, NZC, zinit, 0)
    plsc.subcore_barrier()

    my_nch = jnp.where(c == 0, NCH0, NCH1)
    ebase = s * (NCH0 + NCH1) * CH + c * (NCH0 * CH)

    pltpu.sync_copy(src_hbm.at[pl.ds(ebase, CH)], sidx0)
    pltpu.sync_copy(dst_hbm.at[pl.ds(ebase, CH)], didx0)
    pltpu.async_copy(y_hbm.at[sidx0], rows0, sem0)

    def body(i, _):
        eb = ebase + 2 * i * CH
        pltpu.sync_copy(src_hbm.at[pl.ds(eb + CH, CH)], sidx1)
        pltpu.sync_copy(dst_hbm.at[pl.ds(eb + CH, CH)], didx1)
        pltpu.async_copy(y_hbm.at[sidx1], rows1, sem1)
        pltpu.make_async_copy(y_hbm.at[sidx0], rows0, sem0).wait()
        pltpu.sync_copy(rows0, acc_sh.at[didx0], add=True)
        pltpu.sync_copy(src_hbm.at[pl.ds(eb + 2 * CH, CH)], sidx0)
        pltpu.sync_copy(dst_hbm.at[pl.ds(eb + 2 * CH, CH)], didx0)
        pltpu.async_copy(y_hbm.at[sidx0], rows0, sem0)
        pltpu.make_async_copy(y_hbm.at[sidx1], rows1, sem1).wait()
        pltpu.sync_copy(rows1, acc_sh.at[didx1], add=True)
        return 0

    lax.fori_loop(0, my_nch // 2, body, 0)
    pltpu.make_async_copy(y_hbm.at[sidx0], rows0, sem0).wait()
    plsc.subcore_barrier()

    def wb(k, _):
        r = base_r + k * CH
        pltpu.sync_copy(acc_sh.at[pl.ds(r, CH)],
                        out_hbm.at[pl.ds(c * NPAD + r, CH)])
        return 0

    lax.fori_loop(0, NZC, wb, 0)


def _dinv_block(deg_ref, i):
    deg = deg_ref[0, :, 0:1] + deg_ref[1, :, 0:1] + 1.0
    rows = i * BLK + lax.broadcasted_iota(jnp.int32, (BLK, 1), 0)
    return jnp.where(rows < N, lax.rsqrt(deg), 0.0)


def _mm_scale_body(x_ref, w_ref, deg_ref, y_ref):
    i = pl.program_id(0)
    dinv = _dinv_block(deg_ref, i)
    y_ref[...] = dinv * jnp.dot(x_ref[...], w_ref[...],
                                preferred_element_type=jnp.float32)


def _layer2_body(agg_ref, y_ref, deg_ref, b1_ref, w2_ref, out_ref):
    i = pl.program_id(0)
    dinv = _dinv_block(deg_ref, i)
    h = dinv * (agg_ref[0] + agg_ref[1] + y_ref[...]) + b1_ref[...]
    h = jnp.where(h > 0, h, NEG * h)
    out_ref[...] = dinv * jnp.dot(h, w2_ref[...],
                                  preferred_element_type=jnp.float32)


def _pool_head_body(agg_ref, y_ref, deg_ref, b2_ref, batch_ref,
                    wl_ref, bl_ref, wo_ref, bo_ref,
                    logits_ref, probs_ref, embeds_ref, acc_ref):
    i = pl.program_id(0)

    @pl.when(i == 0)
    def _():
        acc_ref[...] = jnp.full((NG, D), -jnp.inf, jnp.float32)

    dinv = _dinv_block(deg_ref, i)
    h = dinv * (agg_ref[0] + agg_ref[1] + y_ref[...]) + b2_ref[...]

    b = batch_ref[...]
    g_lo = jnp.min(b)
    g_hi = jnp.minimum(jnp.max(b), NG - 1)

    def seg(g, _):
        vals = jnp.where(b == g, h, -jnp.inf)
        m = jnp.max(vals, axis=0, keepdims=True)
        acc_ref[pl.ds(g, 1), :] = jnp.maximum(acc_ref[pl.ds(g, 1), :], m)
        return 0

    lax.fori_loop(g_lo, g_hi + 1, seg, 0)

    @pl.when(i == NB - 1)
    def _():
        pooled = acc_ref[...]
        embeds = jnp.where(pooled == -jnp.inf, 0.0, pooled)
        g1 = jnp.dot(embeds, wl_ref[...],
                     preferred_element_type=jnp.float32) + bl_ref[...]
        g1 = jnp.where(g1 > 0, g1, NEG * g1)
        logits = jnp.dot(g1, wo_ref[...],
                         preferred_element_type=jnp.float32) + bo_ref[...]
        m = jnp.max(logits, axis=-1, keepdims=True)
        ex = jnp.exp(logits - m)
        probs = ex / jnp.sum(ex, axis=-1, keepdims=True)
        logits_ref[...] = logits
        probs_ref[...] = probs
        embeds_ref[...] = embeds


def _mm_scale(x_pad, W1, deg2):
    return pl.pallas_call(
        _mm_scale_body,
        grid=(NB,),
        in_specs=[
            pl.BlockSpec((BLK, D), lambda i: (i, 0)),
            pl.BlockSpec((D, D), lambda i: (0, 0)),
            pl.BlockSpec((2, BLK, DEGW), lambda i: (0, i, 0)),
        ],
        out_specs=pl.BlockSpec((BLK, D), lambda i: (i, 0)),
        out_shape=jax.ShapeDtypeStruct((NPAD, D), jnp.float32),
    )(x_pad, W1, deg2)


def _layer2(agg1, y1, deg2, b1r, W2):
    return pl.pallas_call(
        _layer2_body,
        grid=(NB,),
        in_specs=[
            pl.BlockSpec((2, BLK, D), lambda i: (0, i, 0)),
            pl.BlockSpec((BLK, D), lambda i: (i, 0)),
            pl.BlockSpec((2, BLK, DEGW), lambda i: (0, i, 0)),
            pl.BlockSpec((1, D), lambda i: (0, 0)),
            pl.BlockSpec((D, D), lambda i: (0, 0)),
        ],
        out_specs=pl.BlockSpec((BLK, D), lambda i: (i, 0)),
        out_shape=jax.ShapeDtypeStruct((NPAD, D), jnp.float32),
    )(agg1, y1, deg2, b1r, W2)


def _pool_head(agg2, y2, deg2, b2r, batch_bc, Wl, blr, Wo, bor):
    return pl.pallas_call(
        _pool_head_body,
        grid=(NB,),
        in_specs=[
            pl.BlockSpec((2, BLK, D), lambda i: (0, i, 0)),
            pl.BlockSpec((BLK, D), lambda i: (i, 0)),
            pl.BlockSpec((2, BLK, DEGW), lambda i: (0, i, 0)),
            pl.BlockSpec((1, D), lambda i: (0, 0)),
            pl.BlockSpec((BLK, D), lambda i: (i, 0)),
            pl.BlockSpec((D, D), lambda i: (0, 0)),
            pl.BlockSpec((1, D), lambda i: (0, 0)),
            pl.BlockSpec((D, NC), lambda i: (0, 0)),
            pl.BlockSpec((1, NC), lambda i: (0, 0)),
        ],
        out_specs=[
            pl.BlockSpec((NG, NC), lambda i: (0, 0)),
            pl.BlockSpec((NG, NC), lambda i: (0, 0)),
            pl.BlockSpec((NG, D), lambda i: (0, 0)),
        ],
        out_shape=[
            jax.ShapeDtypeStruct((NG, NC), jnp.float32),
            jax.ShapeDtypeStruct((NG, NC), jnp.float32),
            jax.ShapeDtypeStruct((NG, D), jnp.float32),
        ],
        scratch_shapes=[pltpu.VMEM((NG, D), jnp.float32)],
    )(agg2, y2, deg2, b2r, batch_bc, Wl, blr, Wo, bor)


def kernel(x, edge_index, batch, W1, b1, W2, b2, Wl, bl, Wo, bo):
    src = edge_index[0].astype(jnp.int32)
    dst = edge_index[1].astype(jnp.int32)
    pad_idx = jnp.full((EPAD - E + 2 * CH,), N, jnp.int32)
    srcp = jnp.concatenate([src, pad_idx])
    dstp = jnp.concatenate([dst, pad_idx])
    x_pad = jnp.pad(x, ((0, NPAD - N), (0, 0)))
    batch_pad = jnp.pad(batch.astype(jnp.int32), (0, NPAD - N),
                        constant_values=127)
    batch_bc = jnp.broadcast_to(batch_pad[:, None], (NPAD, D))
    b1r = b1.reshape(1, D)
    b2r = b2.reshape(1, D)
    blr = bl.reshape(1, D)
    bor = bo.reshape(1, NC)

    ones_row = jnp.ones((CH, DEGW), jnp.float32)
    zrow = jnp.zeros((CH, DEGW), jnp.float32)
    zrows = jnp.zeros((CH, D), jnp.float32)

    deg_fn = _build_deg_kernel()
    agg_fn = _build_agg_kernel()
    deg2 = deg_fn(dstp, ones_row, zrow).reshape(2, NPAD, DEGW)
    y1 = _mm_scale(x_pad, W1, deg2)
    agg1 = agg_fn(y1, srcp, dstp, zrows).reshape(2, NPAD, D)
    y2 = _layer2(agg1, y1, deg2, b1r, W2)
    agg2 = agg_fn(y2, srcp, dstp, zrows).reshape(2, NPAD, D)
    logits, probs, embeds = _pool_head(agg2, y2, deg2, b2r, batch_bc,
                                       Wl, blr, Wo, bor)
    return (logits, probs, embeds)


# speedup vs baseline: 9.9698x; 1.1750x over previous
import functools

import jax
import jax.numpy as jnp
from jax import lax
from jax.experimental import pallas as pl
from jax.experimental.pallas import tpu as pltpu
from jax.experimental.pallas import tpu_sc as plsc

N = 10000
NPAD = 10240
D = 128
E = 320000
EPAD = 327680
NW = 32
EPT = EPAD // NW
CH = 128
NCH = EPT // CH
NCH0 = 120
NCH1 = 40
RPT = NPAD // 16
NZC = RPT // CH
DEGW = 128
NG = 64
BLK = 512
NB = NPAD // BLK
NC = 16
NEG = 0.01


@functools.cache
def _build_deg_kernel():
    mesh = plsc.VectorSubcoreMesh(core_axis_name="c", subcore_axis_name="s")
    return pl.kernel(
        _deg_body,
        mesh=mesh,
        out_type=jax.ShapeDtypeStruct((2 * NPAD, DEGW), jnp.float32),
        scratch_types=[
            pltpu.VMEM((CH,), jnp.int32),
            pltpu.VMEM((CH, DEGW), jnp.float32),
            pltpu.VMEM_SHARED((NPAD, DEGW), jnp.float32),
        ],
    )


def _deg_body(dst_hbm, ones_hbm, zrow_hbm, out_hbm, didx, ones_v, acc_sh):
    c = lax.axis_index("c")
    s = lax.axis_index("s")
    wid = s * 2 + c

    pltpu.sync_copy(ones_hbm, ones_v)

    base_r = s * RPT

    def zinit(k, _):
        pltpu.sync_copy(zrow_hbm, acc_sh.at[pl.ds(base_r + k * CH, CH)])
        return 0

    lax.fori_loop(0, NZC, zinit, 0)
    plsc.subcore_barrier()

    ebase = wid * EPT

    def body(ch, _):
        pltpu.sync_copy(dst_hbm.at[pl.ds(ebase + ch * CH, CH)], didx)
        pltpu.sync_copy(ones_v, acc_sh.at[didx], add=True)
        return 0

    lax.fori_loop(0, NCH, body, 0)
    plsc.subcore_barrier()

    def wb(k, _):
        r = base_r + k * CH
        pltpu.sync_copy(acc_sh.at[pl.ds(r, CH)],
                        out_hbm.at[pl.ds(c * NPAD + r, CH)])
        return 0

    lax.fori_loop(0, NZC, wb, 0)


@functools.cache
def _build_agg_kernel():
    mesh = plsc.VectorSubcoreMesh(core_axis_name="c", subcore_axis_name="s")
    return pl.kernel(
        _agg_body,
        mesh=mesh,
        out_type=jax.ShapeDtypeStruct((2 * NPAD, D), jnp.float32),
        scratch_types=[
            pltpu.VMEM((CH,), jnp.int32),
            pltpu.VMEM((CH,), jnp.int32),
            pltpu.VMEM((CH,), jnp.int32),
            pltpu.VMEM((CH,), jnp.int32),
            pltpu.VMEM((CH, D), jnp.float32),
            pltpu.VMEM((CH, D), jnp.float32),
            pltpu.VMEM_SHARED((NPAD, D), jnp.float32),
            pltpu.SemaphoreType.DMA,
            pltpu.SemaphoreType.DMA,
        ],
    )


def _agg_body(y_hbm, src_hbm, dst_hbm, zrows_hbm, out_hbm,
              sidx0, didx0, sidx1, didx1, rows0, rows1, acc_sh, sem0, sem1):
    c = lax.axis_index("c")
    s = lax.axis_index("s")

    base_r = s * RPT

    def zinit(k, _):
        pltpu.sync_copy(zrows_hbm, acc_sh.at[pl.ds(base_r + k * CH, CH)])
        return 0

    lax.fori_loop(0, NZC, zinit, 0)
    plsc.subcore_barrier()

    my_nch = jnp.where(c == 0, NCH0, NCH1)
    ebase = s * (NCH0 + NCH1) * CH + c * (NCH0 * CH)

    pltpu.sync_copy(src_hbm.at[pl.ds(ebase, CH)], sidx0)
    pltpu.sync_copy(dst_hbm.at[pl.ds(ebase, CH)], didx0)
    pltpu.async_copy(y_hbm.at[sidx0], rows0, sem0)

    def body(i, _):
        eb = ebase + 2 * i * CH
        pltpu.sync_copy(src_hbm.at[pl.ds(eb + CH, CH)], sidx1)
        pltpu.sync_copy(dst_hbm.at[pl.ds(eb + CH, CH)], didx1)
        pltpu.async_copy(y_hbm.at[sidx1], rows1, sem1)
        pltpu.make_async_copy(y_hbm.at[sidx0], rows0, sem0).wait()
        pltpu.sync_copy(rows0, acc_sh.at[didx0], add=True)
        pltpu.sync_copy(src_hbm.at[pl.ds(eb + 2 * CH, CH)], sidx0)
        pltpu.sync_copy(dst_hbm.at[pl.ds(eb + 2 * CH, CH)], didx0)
        pltpu.async_copy(y_hbm.at[sidx0], rows0, sem0)
        pltpu.make_async_copy(y_hbm.at[sidx1], rows1, sem1).wait()
        pltpu.sync_copy(rows1, acc_sh.at[didx1], add=True)
        return 0

    lax.fori_loop(0, my_nch // 2, body, 0)
    pltpu.make_async_copy(y_hbm.at[sidx0], rows0, sem0).wait()
    plsc.subcore_barrier()

    def wb(k, _):
        r = base_r + k * CH
        pltpu.sync_copy(acc_sh.at[pl.ds(r, CH)],
                        out_hbm.at[pl.ds(c * NPAD + r, CH)])
        return 0

    lax.fori_loop(0, NZC, wb, 0)


def _dinv_block(deg_ref, i):
    deg = deg_ref[0, :, 0:1] + deg_ref[1, :, 0:1] + 1.0
    rows = i * BLK + lax.broadcasted_iota(jnp.int32, (BLK, 1), 0)
    return jnp.where(rows < N, lax.rsqrt(deg), 0.0)


def _mm_scale_body(x_ref, w_ref, deg_ref, y_ref):
    i = pl.program_id(0)
    dinv = _dinv_block(deg_ref, i)
    y_ref[...] = dinv * jnp.dot(x_ref[...], w_ref[...],
                                preferred_element_type=jnp.float32)


def _layer2_body(agg_ref, y_ref, deg_ref, b1_ref, w2_ref, out_ref):
    i = pl.program_id(0)
    dinv = _dinv_block(deg_ref, i)
    h = dinv * (agg_ref[0] + agg_ref[1] + y_ref[...]) + b1_ref[...]
    h = jnp.where(h > 0, h, NEG * h)
    out_ref[...] = dinv * jnp.dot(h, w2_ref[...],
                                  preferred_element_type=jnp.float32)


def _pool_head_body(agg_ref, y_ref, deg_ref, b2_ref, batch_ref,
                    wl_ref, bl_ref, wo_ref, bo_ref,
                    logits_ref, probs_ref, embeds_ref, acc_ref):
    i = pl.program_id(0)

    @pl.when(i == 0)
    def _():
        acc_ref[...] = jnp.full((NG, D), -jnp.inf, jnp.float32)

    dinv = _dinv_block(deg_ref, i)
    h = dinv * (agg_ref[0] + agg_ref[1] + y_ref[...]) + b2_ref[...]

    b = batch_ref[...]
    g_lo = jnp.min(b)
    g_hi = jnp.minimum(jnp.max(b), NG - 1)

    def seg(g, _):
        vals = jnp.where(b == g, h, -jnp.inf)
        m = jnp.max(vals, axis=0, keepdims=True)
        acc_ref[pl.ds(g, 1), :] = jnp.maximum(acc_ref[pl.ds(g, 1), :], m)
        return 0

    lax.fori_loop(g_lo, g_hi + 1, seg, 0)

    @pl.when(i == NB - 1)
    def _():
        pooled = acc_ref[...]
        embeds = jnp.where(pooled == -jnp.inf, 0.0, pooled)
        g1 = jnp.dot(embeds, wl_ref[...],
                     preferred_element_type=jnp.float32) + bl_ref[...]
        g1 = jnp.where(g1 > 0, g1, NEG * g1)
        logits = jnp.dot(g1, wo_ref[...],
                         preferred_element_type=jnp.float32) + bo_ref[...]
        m = jnp.max(logits, axis=-1, keepdims=True)
        ex = jnp.exp(logits - m)
        probs = ex / jnp.sum(ex, axis=-1, keepdims=True)
        logits_ref[...] = logits
        probs_ref[...] = probs
        embeds_ref[...] = embeds


def _mm_scale(x_pad, W1, deg2):
    return pl.pallas_call(
        _mm_scale_body,
        grid=(NB,),
        in_specs=[
            pl.BlockSpec((BLK, D), lambda i: (i, 0)),
            pl.BlockSpec((D, D), lambda i: (0, 0)),
            pl.BlockSpec((2, BLK, DEGW), lambda i: (0, i, 0)),
        ],
        out_specs=pl.BlockSpec((BLK, D), lambda i: (i, 0)),
        out_shape=jax.ShapeDtypeStruct((NPAD, D), jnp.float32),
    )(x_pad, W1, deg2)


def _layer2(agg1, y1, deg2, b1r, W2):
    return pl.pallas_call(
        _layer2_body,
        grid=(NB,),
        in_specs=[
            pl.BlockSpec((2, BLK, D), lambda i: (0, i, 0)),
            pl.BlockSpec((BLK, D), lambda i: (i, 0)),
            pl.BlockSpec((2, BLK, DEGW), lambda i: (0, i, 0)),
            pl.BlockSpec((1, D), lambda i: (0, 0)),
            pl.BlockSpec((D, D), lambda i: (0, 0)),
        ],
        out_specs=pl.BlockSpec((BLK, D), lambda i: (i, 0)),
        out_shape=jax.ShapeDtypeStruct((NPAD, D), jnp.float32),
    )(agg1, y1, deg2, b1r, W2)


def _pool_head(agg2, y2, deg2, b2r, batch_bc, Wl, blr, Wo, bor):
    return pl.pallas_call(
        _pool_head_body,
        grid=(NB,),
        in_specs=[
            pl.BlockSpec((2, BLK, D), lambda i: (0, i, 0)),
            pl.BlockSpec((BLK, D), lambda i: (i, 0)),
            pl.BlockSpec((2, BLK, DEGW), lambda i: (0, i, 0)),
            pl.BlockSpec((1, D), lambda i: (0, 0)),
            pl.BlockSpec((BLK, D), lambda i: (i, 0)),
            pl.BlockSpec((D, D), lambda i: (0, 0)),
            pl.BlockSpec((1, D), lambda i: (0, 0)),
            pl.BlockSpec((D, NC), lambda i: (0, 0)),
            pl.BlockSpec((1, NC), lambda i: (0, 0)),
        ],
        out_specs=[
            pl.BlockSpec((NG, NC), lambda i: (0, 0)),
            pl.BlockSpec((NG, NC), lambda i: (0, 0)),
            pl.BlockSpec((NG, D), lambda i: (0, 0)),
        ],
        out_shape=[
            jax.ShapeDtypeStruct((NG, NC), jnp.float32),
            jax.ShapeDtypeStruct((NG, NC), jnp.float32),
            jax.ShapeDtypeStruct((NG, D), jnp.float32),
        ],
        scratch_shapes=[pltpu.VMEM((NG, D), jnp.float32)],
    )(agg2, y2, deg2, b2r, batch_bc, Wl, blr, Wo, bor)


def kernel(x, edge_index, batch, W1, b1, W2, b2, Wl, bl, Wo, bo):
    src = edge_index[0].astype(jnp.int32)
    dst = edge_index[1].astype(jnp.int32)
    pad_idx = jnp.full((EPAD - E + 2 * CH,), N, jnp.int32)
    srcp = jnp.concatenate([src, pad_idx])
    dstp = jnp.concatenate([dst, pad_idx])
    x_pad = jnp.pad(x, ((0, NPAD - N), (0, 0)))
    batch_pad = jnp.pad(batch.astype(jnp.int32), (0, NPAD - N),
                        constant_values=127)
    batch_bc = jnp.broadcast_to(batch_pad[:, None], (NPAD, D))
    b1r = b1.reshape(1, D)
    b2r = b2.reshape(1, D)
    blr = bl.reshape(1, D)
    bor = bo.reshape(1, NC)

    ones_row = jnp.ones((CH, DEGW), jnp.float32)
    zrow = jnp.zeros((CH, DEGW), jnp.float32)
    zrows = jnp.zeros((CH, D), jnp.float32)

    deg_fn = _build_deg_kernel()
    agg_fn = _build_agg_kernel()
    deg2 = deg_fn(dstp, ones_row, zrow).reshape(2, NPAD, DEGW)
    y1 = _mm_scale(x_pad, W1, deg2)
    agg1 = agg_fn(y1, srcp, dstp, zrows).reshape(2, NPAD, D)
    y2 = _layer2(agg1, y1, deg2, b1r, W2)
    agg2 = agg_fn(y2, srcp, dstp, zrows).reshape(2, NPAD, D)
    logits, probs, embeds = _pool_head(agg2, y2, deg2, b2r, batch_bc,
                                       Wl, blr, Wo, bor)
    return (logits, probs, embeds)


# speedup vs baseline: 10.8696x; 1.0903x over previous
import functools

import jax
import jax.numpy as jnp
from jax import lax
from jax.experimental import pallas as pl
from jax.experimental.pallas import tpu as pltpu
from jax.experimental.pallas import tpu_sc as plsc

N = 10000
NPAD = 10240
D = 128
E = 320000
EPAD = 327680
NW = 32
EPT = EPAD // NW
CH = 128
NCH = EPT // CH
NCH0 = 120
NCH1 = 40
RPT = NPAD // 16
NZC = RPT // CH
DEGW = 128
NG = 64
BLK = 512
NB = NPAD // BLK
NC = 16
NEG = 0.01


@functools.cache
def _build_deg_kernel():
    mesh = plsc.VectorSubcoreMesh(core_axis_name="c", subcore_axis_name="s")
    return pl.kernel(
        _deg_body,
        mesh=mesh,
        out_type=jax.ShapeDtypeStruct((2 * NPAD, DEGW), jnp.float32),
        scratch_types=[
            pltpu.VMEM((CH,), jnp.int32),
            pltpu.VMEM((CH, DEGW), jnp.float32),
            pltpu.VMEM_SHARED((NPAD, DEGW), jnp.float32),
        ],
    )


def _deg_body(dst_hbm, ones_hbm, zrow_hbm, out_hbm, didx, ones_v, acc_sh):
    c = lax.axis_index("c")
    s = lax.axis_index("s")
    wid = s * 2 + c

    pltpu.sync_copy(ones_hbm, ones_v)

    base_r = s * RPT

    def zinit(k, _):
        pltpu.sync_copy(zrow_hbm, acc_sh.at[pl.ds(base_r + k * CH, CH)])
        return 0

    lax.fori_loop(0, NZC, zinit, 0)
    plsc.subcore_barrier()

    ebase = wid * EPT

    def body(ch, _):
        pltpu.sync_copy(dst_hbm.at[pl.ds(ebase + ch * CH, CH)], didx)
        pltpu.sync_copy(ones_v, acc_sh.at[didx], add=True)
        return 0

    lax.fori_loop(0, NCH, body, 0)
    plsc.subcore_barrier()

    def wb(k, _):
        r = base_r + k * CH
        pltpu.sync_copy(acc_sh.at[pl.ds(r, CH)],
                        out_hbm.at[pl.ds(c * NPAD + r, CH)])
        return 0

    lax.fori_loop(0, NZC, wb, 0)


@functools.cache
def _build_agg_kernel():
    mesh = plsc.VectorSubcoreMesh(core_axis_name="c", subcore_axis_name="s")
    return pl.kernel(
        _agg_body,
        mesh=mesh,
        out_type=jax.ShapeDtypeStruct((2 * NPAD, D), jnp.float32),
        scratch_types=[
            pltpu.VMEM((CH,), jnp.int32),
            pltpu.VMEM((CH,), jnp.int32),
            pltpu.VMEM((CH,), jnp.int32),
            pltpu.VMEM((CH,), jnp.int32),
            pltpu.VMEM((CH, D), jnp.float32),
            pltpu.VMEM((CH, D), jnp.float32),
            pltpu.VMEM_SHARED((NPAD, D), jnp.float32),
            pltpu.SemaphoreType.DMA,
            pltpu.SemaphoreType.DMA,
        ],
    )


def _agg_body(y_hbm, src_hbm, dst_hbm, zrows_hbm, out_hbm,
              sidx0, didx0, sidx1, didx1, rows0, rows1, acc_sh, sem0, sem1):
    c = lax.axis_index("c")
    s = lax.axis_index("s")

    base_r = s * RPT

    def zinit(k, _):
        pltpu.sync_copy(zrows_hbm, acc_sh.at[pl.ds(base_r + k * CH, CH)])
        return 0

    lax.fori_loop(0, NZC, zinit, 0)
    plsc.subcore_barrier()

    my_nch = jnp.where(c == 0, NCH0, NCH1)
    ebase = s * (NCH0 + NCH1) * CH + c * (NCH0 * CH)

    pltpu.sync_copy(src_hbm.at[pl.ds(ebase, CH)], sidx0)
    pltpu.sync_copy(dst_hbm.at[pl.ds(ebase, CH)], didx0)
    pltpu.async_copy(y_hbm.at[sidx0], rows0, sem0)

    def body(i, _):
        eb = ebase + 2 * i * CH
        pltpu.sync_copy(src_hbm.at[pl.ds(eb + CH, CH)], sidx1)
        pltpu.sync_copy(dst_hbm.at[pl.ds(eb + CH, CH)], didx1)
        pltpu.async_copy(y_hbm.at[sidx1], rows1, sem1)
        pltpu.make_async_copy(y_hbm.at[sidx0], rows0, sem0).wait()
        pltpu.sync_copy(rows0, acc_sh.at[didx0], add=True)
        pltpu.sync_copy(src_hbm.at[pl.ds(eb + 2 * CH, CH)], sidx0)
        pltpu.sync_copy(dst_hbm.at[pl.ds(eb + 2 * CH, CH)], didx0)
        pltpu.async_copy(y_hbm.at[sidx0], rows0, sem0)
        pltpu.make_async_copy(y_hbm.at[sidx1], rows1, sem1).wait()
        pltpu.sync_copy(rows1, acc_sh.at[didx1], add=True)
        return 0

    lax.fori_loop(0, my_nch // 2, body, 0)
    pltpu.make_async_copy(y_hbm.at[sidx0], rows0, sem0).wait()
    plsc.subcore_barrier()

    def wb(k, _):
        r = base_r + k * CH
        pltpu.sync_copy(acc_sh.at[pl.ds(r, CH)],
                        out_hbm.at[pl.ds(c * NPAD + r, CH)])
        return 0

    lax.fori_loop(0, NZC, wb, 0)


def _dinv_block(deg_ref, i):
    deg = deg_ref[0, :, 0:1] + deg_ref[1, :, 0:1] + 1.0
    rows = i * BLK + lax.broadcasted_iota(jnp.int32, (BLK, 1), 0)
    return jnp.where(rows < N, lax.rsqrt(deg), 0.0)


def _mm_scale_body(x_ref, w_ref, deg_ref, y_ref):
    i = pl.program_id(0)
    dinv = _dinv_block(deg_ref, i)
    y_ref[...] = dinv * jnp.dot(x_ref[...], w_ref[...],
                                preferred_element_type=jnp.float32)


def _layer2_body(agg_ref, y_ref, deg_ref, b1_ref, w2_ref, out_ref):
    i = pl.program_id(0)
    dinv = _dinv_block(deg_ref, i)
    h = dinv * (agg_ref[0] + agg_ref[1] + y_ref[...]) + b1_ref[...]
    h = jnp.where(h > 0, h, NEG * h)
    out_ref[...] = dinv * jnp.dot(h, w2_ref[...],
                                  preferred_element_type=jnp.float32)


def _pool_head_body(agg_ref, y_ref, deg_ref, b2_ref, batch_ref,
                    wl_ref, bl_ref, wo_ref, bo_ref,
                    logits_ref, probs_ref, embeds_ref, acc_ref):
    i = pl.program_id(0)

    @pl.when(i == 0)
    def _():
        acc_ref[...] = jnp.full((NG, D), -jnp.inf, jnp.float32)

    dinv = _dinv_block(deg_ref, i)
    h = dinv * (agg_ref[0] + agg_ref[1] + y_ref[...]) + b2_ref[...]

    b = batch_ref[...]
    g_lo = jnp.min(b)
    g_hi = jnp.minimum(jnp.max(b), NG - 1)

    def seg(g, _):
        vals = jnp.where(b == g, h, -jnp.inf)
        m = jnp.max(vals, axis=0, keepdims=True)
        acc_ref[pl.ds(g, 1), :] = jnp.maximum(acc_ref[pl.ds(g, 1), :], m)
        return 0

    lax.fori_loop(g_lo, g_hi + 1, seg, 0)

    @pl.when(i == NB - 1)
    def _():
        pooled = acc_ref[...]
        embeds = jnp.where(pooled == -jnp.inf, 0.0, pooled)
        g1 = jnp.dot(embeds, wl_ref[...],
                     preferred_element_type=jnp.float32) + bl_ref[...]
        g1 = jnp.where(g1 > 0, g1, NEG * g1)
        logits = jnp.dot(g1, wo_ref[...],
                         preferred_element_type=jnp.float32) + bo_ref[...]
        m = jnp.max(logits, axis=-1, keepdims=True)
        ex = jnp.exp(logits - m)
        probs = ex / jnp.sum(ex, axis=-1, keepdims=True)
        logits_ref[...] = logits
        probs_ref[...] = probs
        embeds_ref[...] = embeds


def _mm_scale(x_pad, W1, deg2):
    return pl.pallas_call(
        _mm_scale_body,
        grid=(NB,),
        in_specs=[
            pl.BlockSpec((BLK, D), lambda i: (i, 0)),
            pl.BlockSpec((D, D), lambda i: (0, 0)),
            pl.BlockSpec((2, BLK, DEGW), lambda i: (0, i, 0)),
        ],
        out_specs=pl.BlockSpec((BLK, D), lambda i: (i, 0)),
        out_shape=jax.ShapeDtypeStruct((NPAD, D), jnp.float32),
    )(x_pad, W1, deg2)


def _layer2(agg1, y1, deg2, b1r, W2):
    return pl.pallas_call(
        _layer2_body,
        grid=(NB,),
        in_specs=[
            pl.BlockSpec((2, BLK, D), lambda i: (0, i, 0)),
            pl.BlockSpec((BLK, D), lambda i: (i, 0)),
            pl.BlockSpec((2, BLK, DEGW), lambda i: (0, i, 0)),
            pl.BlockSpec((1, D), lambda i: (0, 0)),
            pl.BlockSpec((D, D), lambda i: (0, 0)),
        ],
        out_specs=pl.BlockSpec((BLK, D), lambda i: (i, 0)),
        out_shape=jax.ShapeDtypeStruct((NPAD, D), jnp.float32),
    )(agg1, y1, deg2, b1r, W2)


def _pool_head(agg2, y2, deg2, b2r, batch_bc, Wl, blr, Wo, bor):
    return pl.pallas_call(
        _pool_head_body,
        grid=(NB,),
        in_specs=[
            pl.BlockSpec((2, BLK, D), lambda i: (0, i, 0)),
            pl.BlockSpec((BLK, D), lambda i: (i, 0)),
            pl.BlockSpec((2, BLK, DEGW), lambda i: (0, i, 0)),
            pl.BlockSpec((1, D), lambda i: (0, 0)),
            pl.BlockSpec((BLK, D), lambda i: (i, 0)),
            pl.BlockSpec((D, D), lambda i: (0, 0)),
            pl.BlockSpec((1, D), lambda i: (0, 0)),
            pl.BlockSpec((D, NC), lambda i: (0, 0)),
            pl.BlockSpec((1, NC), lambda i: (0, 0)),
        ],
        out_specs=[
            pl.BlockSpec((NG, NC), lambda i: (0, 0)),
            pl.BlockSpec((NG, NC), lambda i: (0, 0)),
            pl.BlockSpec((NG, D), lambda i: (0, 0)),
        ],
        out_shape=[
            jax.ShapeDtypeStruct((NG, NC), jnp.float32),
            jax.ShapeDtypeStruct((NG, NC), jnp.float32),
            jax.ShapeDtypeStruct((NG, D), jnp.float32),
        ],
        scratch_shapes=[pltpu.VMEM((NG, D), jnp.float32)],
    )(agg2, y2, deg2, b2r, batch_bc, Wl, blr, Wo, bor)


def kernel(x, edge_index, batch, W1, b1, W2, b2, Wl, bl, Wo, bo):
    src = edge_index[0].astype(jnp.int32)
    dst = edge_index[1].astype(jnp.int32)
    pad_idx = jnp.full((EPAD - E + 2 * CH,), N, jnp.int32)
    srcp = jnp.concatenate([src, pad_idx])
    dstp = jnp.concatenate([dst, pad_idx])
    x_pad = jnp.pad(x, ((0, NPAD - N), (0, 0)))
    batch_pad = jnp.pad(batch.astype(jnp.int32), (0, NPAD - N),
                        constant_values=127)
    batch_bc = jnp.broadcast_to(batch_pad[:, None], (NPAD, D))
    b1r = b1.reshape(1, D)
    b2r = b2.reshape(1, D)
    blr = bl.reshape(1, D)
    bor = bo.reshape(1, NC)

    ones_row = jnp.ones((CH, DEGW), jnp.float32)
    zrow = jnp.zeros((CH, DEGW), jnp.float32)
    zrows = jnp.zeros((CH, D), jnp.float32)

    pos = jnp.arange(EPAD + 2 * CH, dtype=jnp.int32)
    blk = (pos // CH) % (NCH0 + NCH1)
    srcp = srcp + jnp.where(blk >= NCH0, NPAD, 0).astype(jnp.int32)

    deg_fn = _build_deg_kernel()
    agg_fn = _build_agg_kernel()
    deg2 = deg_fn(dstp, ones_row, zrow).reshape(2, NPAD, DEGW)
    y1 = _mm_scale(x_pad, W1, deg2)
    y1d = jnp.concatenate([y1, y1], axis=0)
    agg1 = agg_fn(y1d, srcp, dstp, zrows).reshape(2, NPAD, D)
    y2 = _layer2(agg1, y1, deg2, b1r, W2)
    y2d = jnp.concatenate([y2, y2], axis=0)
    agg2 = agg_fn(y2d, srcp, dstp, zrows).reshape(2, NPAD, D)
    logits, probs, embeds = _pool_head(agg2, y2, deg2, b2r, batch_bc,
                                       Wl, blr, Wo, bor)
    return (logits, probs, embeds)
